# Initial kernel scaffold; baseline (speedup 1.0000x reference)
import jax
import jax.numpy as jnp
from jax.experimental import pallas as pl


def kernel(text, adj, weight, bias):
    raise NotImplementedError("write your pallas kernel here")


# speedup vs baseline: 2.3731x; 2.3731x over previous
import jax
import jax.numpy as jnp
from jax.experimental import pallas as pl
from jax.experimental.pallas import tpu as pltpu


def _round_up(x: int, m: int) -> int:
    return (x + m - 1) // m * m


def _pad_to(x, target_shape):
    pads = [(0, t - s) for s, t in zip(x.shape, target_shape)]
    if any(p for _, p in pads):
        x = jnp.pad(x, pads)
    return x


def _gcn_body(text_ref, adj_ref, w_ref, b_ref, out_ref, tw_ref):
    @pl.when(pl.program_id(1) == 0)
    def _compute_tw():
        tw = jnp.dot(text_ref[0].astype(jnp.bfloat16),
                     w_ref[...].astype(jnp.bfloat16),
                     preferred_element_type=jnp.float32)
        tw_ref[...] = tw.astype(jnp.bfloat16)

    adj = adj_ref[0]
    agg = jnp.dot(adj.astype(jnp.bfloat16), tw_ref[...],
                  preferred_element_type=jnp.float32)
    denom = jnp.sum(adj, axis=1, keepdims=True) + 1.0
    inv = pl.reciprocal(denom, approx=False)
    out_ref[0] = agg * inv + b_ref[...]


def kernel(text, adj, weight, bias):
    B, N, Fin = text.shape
    Fin_w, Fout = weight.shape
    assert Fin_w == Fin
    assert adj.shape == (B, N, N)
    if bias is None:
        bias = jnp.zeros((Fout,), dtype=weight.dtype)

    FinP = _round_up(Fin, 128)
    FoutP = _round_up(Fout, 128)

    TM = 256 if N > 256 else _round_up(N, 8)
    NrP = _round_up(N, TM)
    NP = _round_up(N, 128)

    text_p = _pad_to(text, (B, NP, FinP))
    adj_p = _pad_to(adj, (B, NrP, NP))
    w_p = _pad_to(weight, (FinP, FoutP))
    b_p = _pad_to(bias.reshape(1, Fout), (1, FoutP))

    grid = (B, NrP // TM)

    flops = 2.0 * B * N * Fin * Fout + 2.0 * B * N * N * Fout
    bytes_accessed = (text_p.size * 4 + adj_p.size * 4 + w_p.size * 4
                      + b_p.size * 4 + B * NrP * FoutP * 4)
    cost = pl.CostEstimate(flops=int(flops), transcendentals=0,
                           bytes_accessed=int(bytes_accessed))

    out_p = pl.pallas_call(
        _gcn_body,
        out_shape=jax.ShapeDtypeStruct((B, NrP, FoutP), text.dtype),
        grid_spec=pltpu.PrefetchScalarGridSpec(
            num_scalar_prefetch=0,
            grid=grid,
            in_specs=[
                pl.BlockSpec((1, NP, FinP), lambda b, i: (b, 0, 0)),
                pl.BlockSpec((1, TM, NP), lambda b, i: (b, i, 0)),
                pl.BlockSpec((FinP, FoutP), lambda b, i: (0, 0)),
                pl.BlockSpec((1, FoutP), lambda b, i: (0, 0)),
            ],
            out_specs=pl.BlockSpec((1, TM, FoutP), lambda b, i: (b, i, 0)),
            scratch_shapes=[
                pltpu.VMEM((NP, FoutP), jnp.bfloat16),
            ],
        ),
        compiler_params=pltpu.CompilerParams(
            dimension_semantics=("parallel", "arbitrary"),
            vmem_limit_bytes=64 << 20,
        ),
        cost_estimate=cost,
    )(text_p, adj_p, w_p, b_p)

    return out_p[:, :N, :Fout]


# speedup vs baseline: 3.1186x; 1.3141x over previous
import jax
import jax.numpy as jnp
from jax.experimental import pallas as pl
from jax.experimental.pallas import tpu as pltpu


def _round_up(x: int, m: int) -> int:
    return (x + m - 1) // m * m


def _pad_to(x, target_shape):
    pads = [(0, t - s) for s, t in zip(x.shape, target_shape)]
    if any(p for _, p in pads):
        x = jnp.pad(x, pads)
    return x


def _gcn_body(text_ref, adj_ref, w_ref, b_ref, out_ref, tw_ref):
    @pl.when(pl.program_id(1) == 0)
    def _compute_tw():
        tw = jnp.dot(text_ref[0].astype(jnp.bfloat16),
                     w_ref[...].astype(jnp.bfloat16),
                     preferred_element_type=jnp.float32)
        tw_ref[...] = tw.astype(jnp.bfloat16)

    adj = adj_ref[0]
    agg = jnp.dot(adj.astype(jnp.bfloat16), tw_ref[...],
                  preferred_element_type=jnp.float32)
    denom = jnp.sum(adj, axis=1, keepdims=True) + 1.0
    inv = pl.reciprocal(denom, approx=False)
    out_ref[0] = agg * inv + b_ref[...]


def kernel(text, adj, weight, bias):
    B, N, Fin = text.shape
    Fin_w, Fout = weight.shape
    assert Fin_w == Fin
    assert adj.shape == (B, N, N)
    if bias is None:
        bias = jnp.zeros((Fout,), dtype=weight.dtype)

    FinP = _round_up(Fin, 128)
    FoutP = _round_up(Fout, 128)

    TM = 512 if N > 512 else _round_up(N, 8)
    NrP = _round_up(N, TM)
    NP = _round_up(N, 128)

    text_p = _pad_to(text, (B, NP, FinP))
    adj_p = _pad_to(adj, (B, NrP, NP))
    w_p = _pad_to(weight, (FinP, FoutP))
    b_p = _pad_to(bias.reshape(1, Fout), (1, FoutP))

    grid = (B, NrP // TM)

    flops = 2.0 * B * N * Fin * Fout + 2.0 * B * N * N * Fout
    bytes_accessed = (text_p.size * 4 + adj_p.size * 4 + w_p.size * 4
                      + b_p.size * 4 + B * NrP * FoutP * 4)
    cost = pl.CostEstimate(flops=int(flops), transcendentals=0,
                           bytes_accessed=int(bytes_accessed))

    out_p = pl.pallas_call(
        _gcn_body,
        out_shape=jax.ShapeDtypeStruct((B, NrP, FoutP), text.dtype),
        grid_spec=pltpu.PrefetchScalarGridSpec(
            num_scalar_prefetch=0,
            grid=grid,
            in_specs=[
                pl.BlockSpec((1, NP, FinP), lambda b, i: (b, 0, 0)),
                pl.BlockSpec((1, TM, NP), lambda b, i: (b, i, 0)),
                pl.BlockSpec((FinP, FoutP), lambda b, i: (0, 0)),
                pl.BlockSpec((1, FoutP), lambda b, i: (0, 0)),
            ],
            out_specs=pl.BlockSpec((1, TM, FoutP), lambda b, i: (b, i, 0)),
            scratch_shapes=[
                pltpu.VMEM((NP, FoutP), jnp.bfloat16),
            ],
        ),
        compiler_params=pltpu.CompilerParams(
            dimension_semantics=("parallel", "arbitrary"),
            vmem_limit_bytes=64 << 20,
        ),
        cost_estimate=cost,
    )(text_p, adj_p, w_p, b_p)

    return out_p[:, :N, :Fout]


# speedup vs baseline: 3.6289x; 1.1636x over previous
import jax
import jax.numpy as jnp
from jax.experimental import pallas as pl
from jax.experimental.pallas import tpu as pltpu


def _round_up(x: int, m: int) -> int:
    return (x + m - 1) // m * m


def _pad_to(x, target_shape):
    pads = [(0, t - s) for s, t in zip(x.shape, target_shape)]
    if any(p for _, p in pads):
        x = jnp.pad(x, pads)
    return x


def _gcn_body(text_ref, adj_ref, w_ref, b_ref, out_ref, tw_ref):
    @pl.when(pl.program_id(1) == 0)
    def _compute_tw():
        tw = jnp.dot(text_ref[0].astype(jnp.bfloat16),
                     w_ref[...].astype(jnp.bfloat16),
                     preferred_element_type=jnp.float32)
        tw_ref[...] = tw.astype(jnp.bfloat16)

    adj = adj_ref[0]
    agg = jnp.dot(adj.astype(jnp.bfloat16), tw_ref[...],
                  preferred_element_type=jnp.float32)
    denom = jnp.sum(adj, axis=1, keepdims=True) + 1.0
    inv = pl.reciprocal(denom, approx=False)
    out_ref[0] = agg * inv + b_ref[...]


def kernel(text, adj, weight, bias):
    B, N, Fin = text.shape
    Fin_w, Fout = weight.shape
    assert Fin_w == Fin
    assert adj.shape == (B, N, N)
    if bias is None:
        bias = jnp.zeros((Fout,), dtype=weight.dtype)

    FinP = _round_up(Fin, 128)
    FoutP = _round_up(Fout, 128)

    TM = 1024 if N > 1024 else _round_up(N, 8)
    NrP = _round_up(N, TM)
    NP = _round_up(N, 128)

    text_p = _pad_to(text, (B, NP, FinP))
    adj_p = _pad_to(adj, (B, NrP, NP))
    w_p = _pad_to(weight, (FinP, FoutP))
    b_p = _pad_to(bias.reshape(1, Fout), (1, FoutP))

    grid = (B, NrP // TM)

    flops = 2.0 * B * N * Fin * Fout + 2.0 * B * N * N * Fout
    bytes_accessed = (text_p.size * 4 + adj_p.size * 4 + w_p.size * 4
                      + b_p.size * 4 + B * NrP * FoutP * 4)
    cost = pl.CostEstimate(flops=int(flops), transcendentals=0,
                           bytes_accessed=int(bytes_accessed))

    out_p = pl.pallas_call(
        _gcn_body,
        out_shape=jax.ShapeDtypeStruct((B, NrP, FoutP), text.dtype),
        grid_spec=pltpu.PrefetchScalarGridSpec(
            num_scalar_prefetch=0,
            grid=grid,
            in_specs=[
                pl.BlockSpec((1, NP, FinP), lambda b, i: (b, 0, 0)),
                pl.BlockSpec((1, TM, NP), lambda b, i: (b, i, 0)),
                pl.BlockSpec((FinP, FoutP), lambda b, i: (0, 0)),
                pl.BlockSpec((1, FoutP), lambda b, i: (0, 0)),
            ],
            out_specs=pl.BlockSpec((1, TM, FoutP), lambda b, i: (b, i, 0)),
            scratch_shapes=[
                pltpu.VMEM((NP, FoutP), jnp.bfloat16),
            ],
        ),
        compiler_params=pltpu.CompilerParams(
            dimension_semantics=("parallel", "arbitrary"),
            vmem_limit_bytes=64 << 20,
        ),
        cost_estimate=cost,
    )(text_p, adj_p, w_p, b_p)

    return out_p[:, :N, :Fout]


# speedup vs baseline: 3.7779x; 1.0411x over previous
import jax
import jax.numpy as jnp
from jax.experimental import pallas as pl
from jax.experimental.pallas import tpu as pltpu


def _round_up(x: int, m: int) -> int:
    return (x + m - 1) // m * m


def _pad_to(x, target_shape):
    pads = [(0, t - s) for s, t in zip(x.shape, target_shape)]
    if any(p for _, p in pads):
        x = jnp.pad(x, pads)
    return x


def _gcn_body(text_ref, adj_ref, w_ref, b_ref, out_ref, tw_ref):
    @pl.when(pl.program_id(1) == 0)
    def _compute_tw():
        tw = jnp.dot(text_ref[0].astype(jnp.bfloat16),
                     w_ref[...].astype(jnp.bfloat16),
                     preferred_element_type=jnp.float32)
        tw_ref[...] = tw.astype(jnp.bfloat16)

    adj = adj_ref[0]
    agg = jnp.dot(adj.astype(jnp.bfloat16), tw_ref[...],
                  preferred_element_type=jnp.float32)
    denom = jnp.sum(adj, axis=1, keepdims=True) + 1.0
    inv = pl.reciprocal(denom, approx=False)
    out_ref[0] = agg * inv + b_ref[...]


def kernel(text, adj, weight, bias):
    B, N, Fin = text.shape
    Fin_w, Fout = weight.shape
    assert Fin_w == Fin
    assert adj.shape == (B, N, N)
    if bias is None:
        bias = jnp.zeros((Fout,), dtype=weight.dtype)

    FinP = _round_up(Fin, 128)
    FoutP = _round_up(Fout, 128)

    TM = 2048 if N > 2048 else _round_up(N, 8)
    NrP = _round_up(N, TM)
    NP = _round_up(N, 128)

    text_p = _pad_to(text, (B, NP, FinP))
    adj_p = _pad_to(adj, (B, NrP, NP))
    w_p = _pad_to(weight, (FinP, FoutP))
    b_p = _pad_to(bias.reshape(1, Fout), (1, FoutP))

    grid = (B, NrP // TM)

    flops = 2.0 * B * N * Fin * Fout + 2.0 * B * N * N * Fout
    bytes_accessed = (text_p.size * 4 + adj_p.size * 4 + w_p.size * 4
                      + b_p.size * 4 + B * NrP * FoutP * 4)
    cost = pl.CostEstimate(flops=int(flops), transcendentals=0,
                           bytes_accessed=int(bytes_accessed))

    out_p = pl.pallas_call(
        _gcn_body,
        out_shape=jax.ShapeDtypeStruct((B, NrP, FoutP), text.dtype),
        grid_spec=pltpu.PrefetchScalarGridSpec(
            num_scalar_prefetch=0,
            grid=grid,
            in_specs=[
                pl.BlockSpec((1, NP, FinP), lambda b, i: (b, 0, 0)),
                pl.BlockSpec((1, TM, NP), lambda b, i: (b, i, 0)),
                pl.BlockSpec((FinP, FoutP), lambda b, i: (0, 0)),
                pl.BlockSpec((1, FoutP), lambda b, i: (0, 0)),
            ],
            out_specs=pl.BlockSpec((1, TM, FoutP), lambda b, i: (b, i, 0)),
            scratch_shapes=[
                pltpu.VMEM((NP, FoutP), jnp.bfloat16),
            ],
        ),
        compiler_params=pltpu.CompilerParams(
            dimension_semantics=("parallel", "arbitrary"),
            vmem_limit_bytes=64 << 20,
        ),
        cost_estimate=cost,
    )(text_p, adj_p, w_p, b_p)

    return out_p[:, :N, :Fout]
